# baseline (speedup 1.0000x reference)
import functools

import jax
import jax.numpy as jnp
from jax import lax
from jax.experimental import pallas as pl
from jax.experimental.pallas import tpu as pltpu
from jax.experimental.pallas import tpu_sc as plsc

N = 10000
NP = 10240
E = 320000
D = 128
DH = D // 2

NS = 16
EPT = E // NS
CH = 80
NCH = EPT // CH
RPT = NP // NS
SCH = CH // 2
LAG = 8


_BLK = 1000


def _store_proj(h, asrc_ref, adst_ref, hlo_ref, hhi_ref, s_ref, t_ref):
    hlo_ref[...] = h[:, :DH]
    hhi_ref[...] = h[:, DH:]
    s_ref[...] = jnp.dot(h, asrc_ref[...],
                         preferred_element_type=jnp.float32)
    t_ref[...] = jnp.dot(h, adst_ref[...],
                         preferred_element_type=jnp.float32)


def _tc_proj1_body(x_ref, w_ref, asrc_ref, adst_ref,
                   hlo_ref, hhi_ref, s_ref, t_ref):
    h = jnp.dot(x_ref[...], w_ref[...], preferred_element_type=jnp.float32)
    _store_proj(h, asrc_ref, adst_ref, hlo_ref, hhi_ref, s_ref, t_ref)


def _tc_proj2_body(xlo_ref, xhi_ref, w_ref, asrc_ref, adst_ref,
                   hlo_ref, hhi_ref, s_ref, t_ref):
    h = (jnp.dot(xlo_ref[...], w_ref[:DH, :],
                 preferred_element_type=jnp.float32)
         + jnp.dot(xhi_ref[...], w_ref[DH:, :],
                   preferred_element_type=jnp.float32))
    _store_proj(h, asrc_ref, adst_ref, hlo_ref, hhi_ref, s_ref, t_ref)


_tc_out_shape = [
    jax.ShapeDtypeStruct((N, DH), jnp.float32),
    jax.ShapeDtypeStruct((N, DH), jnp.float32),
    jax.ShapeDtypeStruct((N, 1), jnp.float32),
    jax.ShapeDtypeStruct((N, 1), jnp.float32),
]
_tc_full_specs = [
    pl.BlockSpec((D, D), lambda i: (0, 0)),
    pl.BlockSpec((D, 1), lambda i: (0, 0)),
    pl.BlockSpec((D, 1), lambda i: (0, 0)),
]
_tc_out_specs = [
    pl.BlockSpec((_BLK, DH), lambda i: (i, 0)),
    pl.BlockSpec((_BLK, DH), lambda i: (i, 0)),
    pl.BlockSpec((_BLK, 1), lambda i: (i, 0)),
    pl.BlockSpec((_BLK, 1), lambda i: (i, 0)),
]

_tc_proj1 = pl.pallas_call(
    _tc_proj1_body,
    grid=(N // _BLK,),
    in_specs=[pl.BlockSpec((_BLK, D), lambda i: (i, 0))] + _tc_full_specs,
    out_specs=_tc_out_specs,
    out_shape=_tc_out_shape,
)
_tc_proj2 = pl.pallas_call(
    _tc_proj2_body,
    grid=(N // _BLK,),
    in_specs=[pl.BlockSpec((_BLK, DH), lambda i: (i, 0)),
              pl.BlockSpec((_BLK, DH), lambda i: (i, 0))] + _tc_full_specs,
    out_specs=_tc_out_specs,
    out_shape=_tc_out_shape,
)


_mesh = plsc.VectorSubcoreMesh(core_axis_name="c", subcore_axis_name="s")


@functools.partial(
    pl.kernel,
    out_type=[jax.ShapeDtypeStruct((NP, DH), jnp.float32),
              jax.ShapeDtypeStruct((NP, DH), jnp.float32)],
    mesh=_mesh,
    scratch_types=[
        pltpu.VMEM((NP,), jnp.float32),
        pltpu.VMEM((NP,), jnp.float32),
        pltpu.VMEM((NCH, CH), jnp.int32),
        pltpu.VMEM((NCH, CH), jnp.int32),
        pltpu.VMEM((NCH, CH), jnp.float32),
        pltpu.VMEM((3, SCH, DH), jnp.float32),
        pltpu.VMEM((16,), jnp.float32),
        pltpu.VMEM((NS, 16), jnp.float32),
        pltpu.VMEM_SHARED((NP, DH), jnp.float32),
        pltpu.VMEM_SHARED((NP,), jnp.float32),
        pltpu.VMEM_SHARED((NS, 16), jnp.float32),
        pltpu.SemaphoreType.DMA,
        pltpu.SemaphoreType.DMA,
        pltpu.SemaphoreType.DMA,
        pltpu.SemaphoreType.DMA,
        pltpu.SemaphoreType.DMA,
        pltpu.SemaphoreType.DMA,
        pltpu.SemaphoreType.DMA,
    ],
    compiler_params=pltpu.CompilerParams(needs_layout_passes=False,
                                         use_tc_tiling_on_sc=False),
)
def _sc_gat(s_hbm, t_hbm, srcm_hbm, dstm_hbm, hlo_hbm, hhi_hbm,
            outlo_hbm, outhi_hbm,
            s_v, t_v, src_v, dst_v, e_v, rows3,
            mx_v, mxa_v, acc_sh, den_sh, mx_sh,
            sem_g0, sem_g1, sem_g2, sem_s0, sem_s1, sem_s2, sem_d):
    cid = lax.axis_index("c")
    sid = lax.axis_index("s")
    sems_g = (sem_g0, sem_g1, sem_g2)
    sems_s = (sem_s0, sem_s1, sem_s2)

    zero16 = jnp.zeros((16,), jnp.float32)

    pltpu.async_copy(s_hbm, s_v.at[pl.ds(0, N)], sem_g0)
    pltpu.async_copy(t_hbm, t_v.at[pl.ds(0, N)], sem_g1)
    pltpu.async_copy(srcm_hbm.at[sid], src_v, sem_g2)
    pltpu.async_copy(dstm_hbm.at[sid], dst_v, sem_s0)

    def _zr(r, carry):
        for q in range(DH // 16):
            rows3[0, r, pl.ds(q * 16, 16)] = zero16
        return carry
    lax.fori_loop(0, SCH, _zr, 0)
    for k in range(CH // 16):
        e_v[0, pl.ds(k * 16, 16)] = zero16

    def _za(b, carry):
        pltpu.sync_copy(rows3.at[0], acc_sh.at[pl.ds(sid * RPT + b * SCH,
                                                     SCH)])
        return carry
    lax.fori_loop(0, RPT // SCH, _za, 0)

    def _zd(b, carry):
        pltpu.sync_copy(e_v.at[0], den_sh.at[pl.ds(sid * RPT + b * CH, CH)])
        return carry
    lax.fori_loop(0, RPT // CH, _zd, 0)

    pltpu.make_async_copy(s_hbm, s_v.at[pl.ds(0, N)], sem_g0).wait()
    pltpu.make_async_copy(t_hbm, t_v.at[pl.ds(0, N)], sem_g1).wait()
    pltpu.make_async_copy(srcm_hbm.at[sid], src_v, sem_g2).wait()
    pltpu.make_async_copy(dstm_hbm.at[sid], dst_v, sem_s0).wait()
    plsc.subcore_barrier()

    def _e_body(j, mx):
        for k in range(CH // 16):
            si = src_v[j, pl.ds(k * 16, 16)]
            di = dst_v[j, pl.ds(k * 16, 16)]
            ev = plsc.load_gather(s_v, [si]) + plsc.load_gather(t_v, [di])
            ev = jnp.where(ev >= 0.0, ev, 0.2 * ev)
            e_v[j, pl.ds(k * 16, 16)] = ev
            mx = jnp.maximum(mx, ev)
        return mx
    mx = lax.fori_loop(0, NCH, _e_body,
                       jnp.full((16,), -jnp.inf, jnp.float32))
    mx_v[...] = mx
    pltpu.sync_copy(mx_v, mx_sh.at[sid])
    plsc.subcore_barrier()

    pltpu.sync_copy(mx_sh, mxa_v)
    m16 = mxa_v[0, :]
    for i in range(1, NS):
        m16 = jnp.maximum(m16, mxa_v[i, :])
    mval = jnp.max(m16)
    mvec = jnp.full((16,), mval, jnp.float32)

    def _x_fire(j):
        for k in range(CH // 16):
            ev = e_v[j, pl.ds(k * 16, 16)]
            e_v[j, pl.ds(k * 16, 16)] = jnp.exp(ev - mvec)
        pltpu.async_copy(e_v.at[j], den_sh.at[dst_v.at[j]], sem_d, add=True)

    def _x_wait():
        pltpu.make_async_copy(e_v.at[0], den_sh.at[dst_v.at[0]],
                              sem_d).wait()

    def _x_head(j, carry):
        _x_fire(j)
        return carry
    lax.fori_loop(0, LAG, _x_head, 0)

    def _x_body(j, carry):
        _x_fire(j)
        _x_wait()
        return carry
    lax.fori_loop(LAG, NCH, _x_body, 0)
    for _ in range(LAG):
        _x_wait()
    plsc.subcore_barrier()

    pltpu.sync_copy(den_sh, s_v)

    def _heavy(h_half):
        def gi(j, hb):
            return h_half.at[src_v.at[j, pl.ds(hb * SCH, SCH)]]

        def so(j, hb):
            return acc_sh.at[dst_v.at[j, pl.ds(hb * SCH, SCH)]]

        def issue_g(j, hb, b):
            pltpu.async_copy(gi(j, hb), rows3.at[b], sems_g[b])

        def wait_g(j, hb, b):
            pltpu.make_async_copy(gi(j, hb), rows3.at[b], sems_g[b]).wait()

        def issue_s(j, hb, b):
            pltpu.async_copy(rows3.at[b], so(j, hb), sems_s[b], add=True)

        def wait_s(b):
            pltpu.make_async_copy(rows3.at[b], so(0, 0), sems_s[b]).wait()

        def scale(j, hb, b):
            base = hb * SCH
            blks = sorted({(base + r) // 16 for r in range(SCH)})
            avs = {}
            for blk in blks:
                ex = e_v[j, pl.ds(blk * 16, 16)]
                di = dst_v[j, pl.ds(blk * 16, 16)]
                dv = plsc.load_gather(s_v, [di])
                avs[blk] = ex / (dv + 1e-9)
            for r in range(SCH):
                lane = base + r
                av = jnp.full((16,), avs[lane // 16][lane % 16],
                              jnp.float32)
                for q in range(DH // 16):
                    sl = pl.ds(q * 16, 16)
                    rows3[b, r, sl] = rows3[b, r, sl] * av

        def slot(j, hb, b, first=False, last=False):
            wait_g(j, hb, b)
            scale(j, hb, b)
            issue_s(j, hb, b)
            if not first:
                wait_s((b + 2) % 3)
            if not last:
                issue_g(j + 1, hb, (b + 2) % 3)

        issue_g(0, 0, 0)
        issue_g(0, 1, 1)
        for u in range(6):
            slot(u // 2, u % 2, u % 3, first=(u == 0))

        def grp(g, carry):
            for u in range(6):
                slot(3 * g + u // 2, u % 2, u % 3)
            return carry
        lax.fori_loop(1, 83, grp, 0)

        slot(NCH - 1, 0, 0, last=True)
        slot(NCH - 1, 1, 1, last=True)
        wait_s(1)

    def _epilogue(out_hbm):
        NCK = RPT // SCH

        def r0(c):
            return sid * RPT + c * SCH

        def issue_in(c, b):
            pltpu.async_copy(acc_sh.at[pl.ds(r0(c), SCH)], rows3.at[b],
                             sems_g[b])

        def wait_in(c, b):
            pltpu.make_async_copy(acc_sh.at[pl.ds(r0(c), SCH)],
                                  rows3.at[b], sems_g[b]).wait()

        def issue_out(c, b):
            pltpu.async_copy(rows3.at[b], out_hbm.at[pl.ds(r0(c), SCH)],
                             sems_s[b])

        def wait_out(c, b):
            pltpu.make_async_copy(rows3.at[b],
                                  out_hbm.at[pl.ds(r0(c), SCH)],
                                  sems_s[b]).wait()

        issue_in(0, 0)
        issue_in(1, 1)
        for c in range(NCK):
            b = c % 3
            wait_in(c, b)

            def _elu_row(r, carry2, _b=b):
                for q in range(DH // 16):
                    sl = pl.ds(q * 16, 16)
                    v = rows3[_b, r, sl]
                    rows3[_b, r, sl] = jnp.where(v > 0.0, v,
                                                 jnp.exp(v) - 1.0)
                return carry2
            lax.fori_loop(0, SCH, _elu_row, 0)
            issue_out(c, b)
            if c >= 1:
                wait_out(c - 1, (b + 2) % 3)
            if c + 2 < NCK:
                issue_in(c + 2, (b + 2) % 3)
        wait_out(NCK - 1, (NCK - 1) % 3)

    @pl.when(cid == 0)
    def _():
        _heavy(hlo_hbm)
        plsc.subcore_barrier()
        _epilogue(outlo_hbm)

    @pl.when(cid == 1)
    def _():
        _heavy(hhi_hbm)
        plsc.subcore_barrier()
        _epilogue(outhi_hbm)


def kernel(x, edge_index, W1, a_src1, a_dst1, W2, a_src2, a_dst2):
    src = edge_index[0].astype(jnp.int32)
    dst = edge_index[1].astype(jnp.int32)
    srcm = src.reshape(NS, NCH, CH)
    dstm = dst.reshape(NS, NCH, CH)
    hlo, hhi, s, t = _tc_proj1(x, W1, a_src1[:, None], a_dst1[:, None])
    h1lo, h1hi = _sc_gat(s[:, 0], t[:, 0], srcm, dstm, hlo, hhi)
    hlo2, hhi2, s2, t2 = _tc_proj2(h1lo, h1hi, W2, a_src2[:, None],
                                   a_dst2[:, None])
    h2lo, h2hi = _sc_gat(s2[:, 0], t2[:, 0], srcm, dstm, hlo2, hhi2)
    out = jnp.concatenate([h1lo, h1hi, h2lo, h2hi], axis=1)
    return out[:N].reshape(N, 2, D)


# speedup vs baseline: 1.2735x; 1.2735x over previous
import functools

import jax
import jax.numpy as jnp
from jax import lax
from jax.experimental import pallas as pl
from jax.experimental.pallas import tpu as pltpu
from jax.experimental.pallas import tpu_sc as plsc

N = 10000
NP = 10240
E = 320000
D = 128
DH = D // 2

NS = 16
EPT = E // NS
CH = 80
NCH = EPT // CH
RPT = NP // NS
LAG = 8
DRING = 10


def _store_proj(h, n, asrc_ref, adst_ref, hlo_ref, hhi_ref, s_ref, t_ref):
    hlo_ref[:n, :] = h[:, :DH]
    hhi_ref[:n, :] = h[:, DH:]
    s_ref[:n, :] = jnp.dot(h, asrc_ref[...],
                           preferred_element_type=jnp.float32)
    t_ref[:n, :] = jnp.dot(h, adst_ref[...],
                           preferred_element_type=jnp.float32)


def _tc_proj1_body(x_ref, w_ref, asrc_ref, adst_ref,
                   hlo_ref, hhi_ref, s_ref, t_ref):
    h = jnp.dot(x_ref[...], w_ref[...], preferred_element_type=jnp.float32)
    _store_proj(h, N, asrc_ref, adst_ref, hlo_ref, hhi_ref, s_ref, t_ref)


def _tc_proj2_body(xlo_ref, xhi_ref, w_ref, asrc_ref, adst_ref,
                   hlo_ref, hhi_ref, s_ref, t_ref):
    h = (jnp.dot(xlo_ref[...], w_ref[:DH, :],
                 preferred_element_type=jnp.float32)
         + jnp.dot(xhi_ref[...], w_ref[DH:, :],
                   preferred_element_type=jnp.float32))
    _store_proj(h, NP, asrc_ref, adst_ref, hlo_ref, hhi_ref, s_ref, t_ref)


_tc_out_shape = [
    jax.ShapeDtypeStruct((NP, DH), jnp.float32),
    jax.ShapeDtypeStruct((NP, DH), jnp.float32),
    jax.ShapeDtypeStruct((NP, 1), jnp.float32),
    jax.ShapeDtypeStruct((NP, 1), jnp.float32),
]

_tc_proj1 = pl.pallas_call(_tc_proj1_body, out_shape=_tc_out_shape)
_tc_proj2 = pl.pallas_call(_tc_proj2_body, out_shape=_tc_out_shape)


_mesh = plsc.VectorSubcoreMesh(core_axis_name="c", subcore_axis_name="s")


@functools.partial(
    pl.kernel,
    out_type=[jax.ShapeDtypeStruct((NP, DH), jnp.float32),
              jax.ShapeDtypeStruct((NP, DH), jnp.float32)],
    mesh=_mesh,
    scratch_types=[
        pltpu.VMEM((NP,), jnp.float32),
        pltpu.VMEM((NP,), jnp.float32),
        pltpu.VMEM((NCH, CH), jnp.int32),
        pltpu.VMEM((NCH, CH), jnp.float32),
        pltpu.VMEM((3, CH, DH), jnp.float32),
        pltpu.VMEM((3, CH), jnp.int32),
        pltpu.VMEM((3, CH), jnp.int32),
        pltpu.VMEM((DRING, CH), jnp.int32),
        pltpu.VMEM((16,), jnp.float32),
        pltpu.VMEM((NS, 16), jnp.float32),
        pltpu.VMEM_SHARED((NP, DH), jnp.float32),
        pltpu.VMEM_SHARED((NP,), jnp.float32),
        pltpu.VMEM_SHARED((NS, 16), jnp.float32),
        pltpu.SemaphoreType.DMA,
        pltpu.SemaphoreType.DMA,
        pltpu.SemaphoreType.DMA,
        pltpu.SemaphoreType.DMA,
        pltpu.SemaphoreType.DMA,
        pltpu.SemaphoreType.DMA,
        pltpu.SemaphoreType.DMA,
    ],
    compiler_params=pltpu.CompilerParams(needs_layout_passes=False,
                                         use_tc_tiling_on_sc=False),
)
def _sc_gat(s_hbm, t_hbm, pkm_hbm, hlo_hbm, hhi_hbm,
            outlo_hbm, outhi_hbm,
            s_v, t_v, pk_v, e_v, rows3, gsrc, gdst, dring,
            mx_v, mxa_v, acc_sh, den_sh, mx_sh,
            sem_g0, sem_g1, sem_g2, sem_s0, sem_s1, sem_s2, sem_d):
    cid = lax.axis_index("c")
    sid = lax.axis_index("s")
    sems_g = (sem_g0, sem_g1, sem_g2)
    sems_s = (sem_s0, sem_s1, sem_s2)

    zero16 = jnp.zeros((16,), jnp.float32)

    pltpu.async_copy(s_hbm, s_v, sem_g0)
    pltpu.async_copy(t_hbm, t_v, sem_g1)
    pltpu.async_copy(pkm_hbm.at[sid], pk_v, sem_g2)

    def _zr(r, carry):
        for q in range(DH // 16):
            rows3[0, r, pl.ds(q * 16, 16)] = zero16
        return carry
    lax.fori_loop(0, CH, _zr, 0)
    for k in range(CH // 16):
        e_v[0, pl.ds(k * 16, 16)] = zero16

    def _za(b, carry):
        pltpu.sync_copy(rows3.at[0], acc_sh.at[pl.ds(sid * RPT + b * CH,
                                                     CH)])
        return carry
    lax.fori_loop(0, RPT // CH, _za, 0)

    def _zd(b, carry):
        pltpu.sync_copy(e_v.at[0], den_sh.at[pl.ds(sid * RPT + b * CH, CH)])
        return carry
    lax.fori_loop(0, RPT // CH, _zd, 0)

    pltpu.make_async_copy(s_hbm, s_v, sem_g0).wait()
    pltpu.make_async_copy(t_hbm, t_v, sem_g1).wait()
    pltpu.make_async_copy(pkm_hbm.at[sid], pk_v, sem_g2).wait()
    plsc.subcore_barrier()

    mask16 = jnp.full((16,), 0xFFFF, jnp.int32)
    shift16 = jnp.full((16,), 16, jnp.int32)

    def _unpack(j, k):
        p = pk_v[j, pl.ds(k * 16, 16)]
        return (jnp.bitwise_and(p, mask16),
                lax.shift_right_logical(p, shift16))

    def _e_body(j, mx):
        for k in range(CH // 16):
            si, di = _unpack(j, k)
            ev = plsc.load_gather(s_v, [si]) + plsc.load_gather(t_v, [di])
            ev = jnp.where(ev >= 0.0, ev, 0.2 * ev)
            e_v[j, pl.ds(k * 16, 16)] = ev
            mx = jnp.maximum(mx, ev)
        return mx
    mx = lax.fori_loop(0, NCH, _e_body,
                       jnp.full((16,), -jnp.inf, jnp.float32))
    mx_v[...] = mx
    pltpu.sync_copy(mx_v, mx_sh.at[sid])
    plsc.subcore_barrier()

    pltpu.sync_copy(mx_sh, mxa_v)
    m16 = mxa_v[0, :]
    for i in range(1, NS):
        m16 = jnp.maximum(m16, mxa_v[i, :])
    mval = jnp.max(m16)
    mvec = jnp.full((16,), mval, jnp.float32)

    def _x_fire(j, u):
        for k in range(CH // 16):
            ev = e_v[j, pl.ds(k * 16, 16)]
            e_v[j, pl.ds(k * 16, 16)] = jnp.exp(ev - mvec)
            _, di = _unpack(j, k)
            dring[u, pl.ds(k * 16, 16)] = di
        pltpu.async_copy(e_v.at[j], den_sh.at[dring.at[u]], sem_d, add=True)

    def _x_wait():
        pltpu.make_async_copy(e_v.at[0], den_sh.at[dring.at[0]],
                              sem_d).wait()

    def _x_grp(p, carry):
        for u in range(DRING):
            j = p * DRING + u
            _x_fire(j, u)

            @pl.when(j >= LAG)
            def _():
                _x_wait()
        return carry
    lax.fori_loop(0, NCH // DRING, _x_grp, 0)
    for _ in range(LAG):
        _x_wait()
    plsc.subcore_barrier()

    pltpu.sync_copy(den_sh, s_v)

    def _heavy(h_half):
        def unpack_src(j, b):
            for k in range(CH // 16):
                p = pk_v[j, pl.ds(k * 16, 16)]
                gsrc[b, pl.ds(k * 16, 16)] = jnp.bitwise_and(p, mask16)

        def unpack_dst(j, b):
            for k in range(CH // 16):
                p = pk_v[j, pl.ds(k * 16, 16)]
                gdst[b, pl.ds(k * 16, 16)] = lax.shift_right_logical(
                    p, shift16)

        def issue_g(j, b):
            unpack_src(j, b)
            pltpu.async_copy(h_half.at[gsrc.at[b]], rows3.at[b], sems_g[b])

        def wait_g(b):
            pltpu.make_async_copy(h_half.at[gsrc.at[b]], rows3.at[b],
                                  sems_g[b]).wait()

        def issue_s(j, b):
            unpack_dst(j, b)
            pltpu.async_copy(rows3.at[b], acc_sh.at[gdst.at[b]],
                             sems_s[b], add=True)

        def wait_s(b):
            pltpu.make_async_copy(rows3.at[b], acc_sh.at[gdst.at[b]],
                                  sems_s[b]).wait()

        def scale(j, b):
            avs = {}
            for blk in range(CH // 16):
                ex = e_v[j, pl.ds(blk * 16, 16)]
                _, di = _unpack(j, blk)
                dv = plsc.load_gather(s_v, [di])
                avs[blk] = ex / (dv + 1e-9)
            for r in range(CH):
                av = jnp.full((16,), avs[r // 16][r % 16], jnp.float32)
                for q in range(DH // 16):
                    sl = pl.ds(q * 16, 16)
                    rows3[b, r, sl] = rows3[b, r, sl] * av

        issue_g(0, 0)
        issue_g(1, 1)

        def grp(g, carry):
            for u in range(3):
                j = 3 * g + u
                b = u
                wait_g(b)
                scale(j, b)
                issue_s(j, b)

                @pl.when(j >= 1)
                def _():
                    wait_s((b + 2) % 3)

                @pl.when(j + 2 < NCH)
                def _():
                    issue_g(j + 2, (b + 2) % 3)
            return carry
        lax.fori_loop(0, (NCH - 1) // 3, grp, 0)

        wait_g(0)
        scale(NCH - 1, 0)
        issue_s(NCH - 1, 0)
        wait_s(2)
        wait_s(0)

    def _epilogue(out_hbm):
        NCK = RPT // CH

        def r0(c):
            return sid * RPT + c * CH

        def issue_in(c, b):
            pltpu.async_copy(acc_sh.at[pl.ds(r0(c), CH)], rows3.at[b],
                             sems_g[b])

        def wait_in(c, b):
            pltpu.make_async_copy(acc_sh.at[pl.ds(r0(c), CH)],
                                  rows3.at[b], sems_g[b]).wait()

        def issue_out(c, b):
            pltpu.async_copy(rows3.at[b], out_hbm.at[pl.ds(r0(c), CH)],
                             sems_s[b])

        def wait_out(c, b):
            pltpu.make_async_copy(rows3.at[b],
                                  out_hbm.at[pl.ds(r0(c), CH)],
                                  sems_s[b]).wait()

        issue_in(0, 0)
        issue_in(1, 1)
        for c in range(NCK):
            b = c % 3
            wait_in(c, b)

            def _elu_row(r, carry2, _b=b):
                for q in range(DH // 16):
                    sl = pl.ds(q * 16, 16)
                    v = rows3[_b, r, sl]
                    rows3[_b, r, sl] = jnp.where(v > 0.0, v,
                                                 jnp.exp(v) - 1.0)
                return carry2
            lax.fori_loop(0, CH, _elu_row, 0)
            issue_out(c, b)
            if c >= 1:
                wait_out(c - 1, (b + 2) % 3)
            if c + 2 < NCK:
                issue_in(c + 2, (b + 2) % 3)
        wait_out(NCK - 1, (NCK - 1) % 3)

    @pl.when(cid == 0)
    def _():
        _heavy(hlo_hbm)
        plsc.subcore_barrier()
        _epilogue(outlo_hbm)

    @pl.when(cid == 1)
    def _():
        _heavy(hhi_hbm)
        plsc.subcore_barrier()
        _epilogue(outhi_hbm)


def kernel(x, edge_index, W1, a_src1, a_dst1, W2, a_src2, a_dst2):
    src = edge_index[0].astype(jnp.int32)
    dst = edge_index[1].astype(jnp.int32)
    pkm = (dst * 65536 + src).reshape(NS, NCH, CH)
    hlo, hhi, s, t = _tc_proj1(x, W1, a_src1[:, None], a_dst1[:, None])
    h1lo, h1hi = _sc_gat(s[:, 0], t[:, 0], pkm, hlo, hhi)
    hlo2, hhi2, s2, t2 = _tc_proj2(h1lo, h1hi, W2, a_src2[:, None],
                                   a_dst2[:, None])
    h2lo, h2hi = _sc_gat(s2[:, 0], t2[:, 0], pkm, hlo2, hhi2)
    out = jnp.concatenate([h1lo, h1hi, h2lo, h2hi], axis=1)
    return out[:N].reshape(N, 2, D)
